# baseline (speedup 1.0000x reference)
import jax
import jax.numpy as jnp
from jax import lax
from jax.experimental import pallas as pl
from jax.experimental.pallas import tpu as pltpu
from jax.experimental.pallas import tpu_sc as plsc

NC, NS = 2, 16
NW = NC * NS
CHUNK = 128
D = 64


def _gather_body(table_hbm, idx_hbm, out_hbm, idx_v, rows_v, gsem):
    wid = lax.axis_index("s") * NC + lax.axis_index("c")
    nch = idx_hbm.shape[1]
    per_w = nch * CHUNK
    base = wid * per_w

    pltpu.sync_copy(idx_hbm.at[wid], idx_v)

    @pl.loop(0, nch)
    def _(j):
        pltpu.async_copy(table_hbm.at[idx_v.at[j]], rows_v, gsem).wait()
        pltpu.sync_copy(rows_v, out_hbm.at[pl.ds(base + j * CHUNK, CHUNK)])


def kernel(token_ids, table):
    bt, s = token_ids.shape
    b = bt * s
    d = table.shape[1]
    nch = b // (NW * CHUNK)
    idx = token_ids.reshape(NW, nch, CHUNK).astype(jnp.int32)

    mesh = plsc.VectorSubcoreMesh(core_axis_name="c", subcore_axis_name="s")
    run = pl.kernel(
        _gather_body,
        out_type=jax.ShapeDtypeStruct((b, d), table.dtype),
        mesh=mesh,
        scratch_types=[
            pltpu.VMEM((nch, CHUNK), jnp.int32),
            pltpu.VMEM((CHUNK, D), jnp.float32),
            pltpu.SemaphoreType.DMA,
        ],
        compiler_params=pltpu.CompilerParams(use_tc_tiling_on_sc=False),
    )
    out = run(table, idx)
    return out.reshape(bt, s, d)


# speedup vs baseline: 1.1168x; 1.1168x over previous
import jax
import jax.numpy as jnp
from jax import lax
from jax.experimental import pallas as pl
from jax.experimental.pallas import tpu as pltpu
from jax.experimental.pallas import tpu_sc as plsc

NC, NS = 2, 16
NW = NC * NS
CHUNK = 128
K = 2
GROUP = K * CHUNK
NBUF = 4
D = 64


def _gather_body(table_hbm, idx_hbm, out_hbm, idx_v,
                 b0, b1, b2, b3, g0, g1, g2, g3, s0, s1, s2, s3):
    bufs = (b0, b1, b2, b3)
    gsems = (g0, g1, g2, g3)
    ssems = (s0, s1, s2, s3)

    wid = lax.axis_index("s") * NC + lax.axis_index("c")
    nch = idx_hbm.shape[1]
    ng = nch // K
    base = wid * (nch * CHUNK)

    pltpu.sync_copy(idx_hbm.at[wid], idx_v)

    def fire_gathers(g, p):
        for k in range(K):
            pltpu.async_copy(table_hbm.at[idx_v.at[g * K + k]],
                             bufs[p].at[pl.ds(k * CHUNK, CHUNK)], gsems[p])

    def drain_gathers(p):
        pltpu.make_async_copy(table_hbm.at[pl.ds(0, GROUP)], bufs[p],
                              gsems[p]).wait()

    def fire_store(g, p):
        pltpu.async_copy(bufs[p], out_hbm.at[pl.ds(base + g * GROUP, GROUP)],
                         ssems[p])

    def drain_store(p):
        pltpu.make_async_copy(bufs[p], out_hbm.at[pl.ds(base, GROUP)],
                              ssems[p]).wait()

    fire_gathers(0, 0)
    fire_gathers(1, 1)

    @pl.loop(0, ng, step=NBUF)
    def _(gg):
        for p in range(NBUF):
            g = gg + p
            drain_gathers(p)
            fire_store(g, p)
            q = (p + 2) % NBUF

            @pl.when(g + 2 < ng)
            def _():
                @pl.when(g >= 2)
                def _():
                    drain_store(q)
                fire_gathers(g + 2, q)

    for p in range(NBUF):
        drain_store(p)


def kernel(token_ids, table):
    bt, s = token_ids.shape
    b = bt * s
    d = table.shape[1]
    nch = b // (NW * CHUNK)
    idx = token_ids.reshape(NW, nch, CHUNK).astype(jnp.int32)

    mesh = plsc.VectorSubcoreMesh(core_axis_name="c", subcore_axis_name="s")
    run = pl.kernel(
        _gather_body,
        out_type=jax.ShapeDtypeStruct((b, d), table.dtype),
        mesh=mesh,
        scratch_types=(
            [pltpu.VMEM((nch, CHUNK), jnp.int32)]
            + [pltpu.VMEM((GROUP, D), jnp.float32) for _ in range(NBUF)]
            + [pltpu.SemaphoreType.DMA for _ in range(2 * NBUF)]
        ),
        compiler_params=pltpu.CompilerParams(use_tc_tiling_on_sc=False),
    )
    out = run(table, idx)
    return out.reshape(bt, s, d)


# speedup vs baseline: 1.1179x; 1.0010x over previous
import jax
import jax.numpy as jnp
from jax import lax
from jax.experimental import pallas as pl
from jax.experimental.pallas import tpu as pltpu
from jax.experimental.pallas import tpu_sc as plsc

NC, NS = 2, 16
NW = NC * NS
CHUNK = 512
NBUF = 2
D = 64


def _gather_body(table_hbm, idx_hbm, out_hbm, idx_v, b0, b1, g0, g1, s0, s1):
    bufs = (b0, b1)
    gsems = (g0, g1)
    ssems = (s0, s1)

    wid = lax.axis_index("s") * NC + lax.axis_index("c")
    nch = idx_hbm.shape[1]
    base = wid * (nch * CHUNK)

    pltpu.sync_copy(idx_hbm.at[wid], idx_v)

    def fire_gather(g, p):
        pltpu.async_copy(table_hbm.at[idx_v.at[g]], bufs[p], gsems[p])

    def drain_gather(p):
        pltpu.make_async_copy(table_hbm.at[pl.ds(0, CHUNK)], bufs[p],
                              gsems[p]).wait()

    def fire_store(g, p):
        pltpu.async_copy(bufs[p], out_hbm.at[pl.ds(base + g * CHUNK, CHUNK)],
                         ssems[p])

    def drain_store(p):
        pltpu.make_async_copy(bufs[p], out_hbm.at[pl.ds(base, CHUNK)],
                              ssems[p]).wait()

    fire_gather(0, 0)

    @pl.loop(0, nch, step=NBUF)
    def _(gg):
        for p in range(NBUF):
            g = gg + p
            q = 1 - p

            @pl.when(g + 1 < nch)
            def _():
                @pl.when(g >= 1)
                def _():
                    drain_store(q)
                fire_gather(g + 1, q)

            drain_gather(p)
            fire_store(g, p)

    drain_store(0)
    drain_store(1)


def kernel(token_ids, table):
    bt, s = token_ids.shape
    b = bt * s
    d = table.shape[1]
    nch = b // (NW * CHUNK)
    idx = token_ids.reshape(NW, nch, CHUNK).astype(jnp.int32)

    mesh = plsc.VectorSubcoreMesh(core_axis_name="c", subcore_axis_name="s")
    run = pl.kernel(
        _gather_body,
        out_type=jax.ShapeDtypeStruct((b, d), table.dtype),
        mesh=mesh,
        scratch_types=(
            [pltpu.VMEM((nch, CHUNK), jnp.int32)]
            + [pltpu.VMEM((CHUNK, D), jnp.float32) for _ in range(NBUF)]
            + [pltpu.SemaphoreType.DMA for _ in range(2 * NBUF)]
        ),
        compiler_params=pltpu.CompilerParams(use_tc_tiling_on_sc=False),
    )
    out = run(table, idx)
    return out.reshape(bt, s, d)


# speedup vs baseline: 1.1761x; 1.0521x over previous
import jax
import jax.numpy as jnp
from jax import lax
from jax.experimental import pallas as pl
from jax.experimental.pallas import tpu as pltpu
from jax.experimental.pallas import tpu_sc as plsc

NC, NS = 2, 16
NW = NC * NS
CHUNK = 128
D = 64


def _body(table_hbm, idx_hbm, out_hbm,
          idx_v, p0, p1, r0, r1, c0, c1, g0, g1, s0, s1):
    pidx = (p0, p1)
    rows = (r0, r1)
    cmp = (c0, c1)
    gsems = (g0, g1)
    ssems = (s0, s1)

    wid = lax.axis_index("s") * NC + lax.axis_index("c")
    nch = idx_hbm.shape[1]
    base = wid * (nch * CHUNK)

    pltpu.sync_copy(idx_hbm.at[wid], idx_v)

    def fire_gather(j, p):
        @pl.loop(0, CHUNK // 16)
        def _(g):
            v = idx_v[j, pl.ds(g * 16, 16)]
            pidx[p][pl.ds(g * 16, 16)] = v >> 1
        pltpu.async_copy(table_hbm.at[pidx[p]], rows[p], gsems[p])

    def drain_gather(p):
        pltpu.make_async_copy(table_hbm.at[pl.ds(0, CHUNK)], rows[p],
                              gsems[p]).wait()

    def select(j, p):
        @pl.loop(0, CHUNK // 16)
        def _(g):
            v16 = idx_v[j, pl.ds(g * 16, 16)]
            par16 = (v16 & 1) * 64
            for l in range(16):
                i = g * 16 + l
                col0 = par16[l]
                for k in range(4):
                    cmp[p][i, pl.ds(k * 16, 16)] = \
                        rows[p][i, pl.ds(col0 + k * 16, 16)]

    def fire_store(j, p):
        pltpu.async_copy(cmp[p], out_hbm.at[pl.ds(base + j * CHUNK, CHUNK)],
                         ssems[p])

    def drain_store(p):
        pltpu.make_async_copy(cmp[p], out_hbm.at[pl.ds(base, CHUNK)],
                              ssems[p]).wait()

    fire_gather(0, 0)

    @pl.loop(0, nch, step=2)
    def _(jj):
        for p in range(2):
            j = jj + p

            @pl.when(j + 1 < nch)
            def _():
                fire_gather(j + 1, 1 - p)

            drain_gather(p)

            @pl.when(j >= 2)
            def _():
                drain_store(p)

            select(j, p)
            fire_store(j, p)

    drain_store(0)
    drain_store(1)


def kernel(token_ids, table):
    bt, s = token_ids.shape
    b = bt * s
    nch = b // (NW * CHUNK)
    idx = token_ids.reshape(NW, nch, CHUNK).astype(jnp.int32)
    table_pairs = table.reshape(table.shape[0] // 2, 2 * D)

    mesh = plsc.VectorSubcoreMesh(core_axis_name="c", subcore_axis_name="s")
    run = pl.kernel(
        _body,
        out_type=jax.ShapeDtypeStruct((b, D), table.dtype),
        mesh=mesh,
        scratch_types=(
            [pltpu.VMEM((nch, CHUNK), jnp.int32)]
            + [pltpu.VMEM((CHUNK,), jnp.int32) for _ in range(2)]
            + [pltpu.VMEM((CHUNK, 2 * D), jnp.float32) for _ in range(2)]
            + [pltpu.VMEM((CHUNK, D), jnp.float32) for _ in range(2)]
            + [pltpu.SemaphoreType.DMA for _ in range(4)]
        ),
        compiler_params=pltpu.CompilerParams(use_tc_tiling_on_sc=True),
    )
    out = run(table_pairs, idx)
    return out.reshape(bt, s, D)
